# baseline (speedup 1.0000x reference)
import functools

import jax
import jax.numpy as jnp
from jax.experimental import pallas as pl
from jax.experimental.pallas import tpu as pltpu

_VMEM_LIMIT = 48 * 1024 * 1024
_B1 = 4
_B2 = 4


def _conv_stats_kernel(x_ref, w_ref, y_ref, stats_ref, slab_ref, *, k, h, w):
    rows = h * w
    ones8 = jnp.ones((8, rows), jnp.float32)
    s_tot = jnp.zeros((8, w_ref.shape[1]), jnp.float32)
    q_tot = jnp.zeros((8, w_ref.shape[1]), jnp.float32)
    slab_ref[...] = jnp.zeros_like(slab_ref)
    for b in range(x_ref.shape[0]):
        slab_ref[k // 2:k // 2 + h, k // 2:k // 2 + w, :] = x_ref[b]
        slab = slab_ref[...]
        cols = [slab[di:di + h, dj:dj + w, :].reshape(rows, -1)
                for di in range(k) for dj in range(k)]
        xcol = jnp.concatenate(cols, axis=1)
        acc = jnp.dot(xcol, w_ref[...],
                      preferred_element_type=jnp.float32)
        s_tot = s_tot + jnp.dot(ones8, acc,
                                preferred_element_type=jnp.float32)
        q_tot = q_tot + jnp.dot(ones8, acc * acc,
                                preferred_element_type=jnp.float32)
        y_ref[b] = acc.T.astype(jnp.bfloat16)
    stats_ref[0] = jnp.concatenate([s_tot, q_tot], axis=0)


def _bn_mish_kernel(y_ref, stats_ref, gamma_ref, beta_ref, o_ref,
                    *, count, eps):
    s = jnp.sum(stats_ref[:, 0, :], axis=0)
    q = jnp.sum(stats_ref[:, 8, :], axis=0)
    mean = s / count
    var = jnp.maximum(q / count - mean * mean, 0.0)
    inv_std = jax.lax.rsqrt(var + eps)
    g = gamma_ref[0]
    scale = (g * inv_std).reshape(1, -1, 1)
    shift = (beta_ref[0] - mean * g * inv_std).reshape(1, -1, 1)
    z = y_ref[...].astype(jnp.float32) * scale + shift
    t = jnp.exp(jnp.minimum(z, 20.0))
    u = t * (t + 2.0)
    mish = z * u * pl.reciprocal(u + 2.0, approx=True)
    o_ref[...] = jnp.where(z > 20.0, z, mish).astype(o_ref.dtype)


def kernel(x_nchw, conv_w, gamma, beta):
    eps = 1e-5
    N, Cin, H, W = x_nchw.shape
    Cout, _, k, _ = conv_w.shape
    p = k // 2
    rows = H * W
    Hp, Wp = H + 2 * p, W + 2 * p
    b1 = _B1 if N % _B1 == 0 else 1
    b2 = _B2 if N % _B2 == 0 else 1

    xp = jnp.transpose(x_nchw, (0, 2, 3, 1)).astype(jnp.bfloat16)

    w_flat = jnp.transpose(conv_w, (2, 3, 1, 0)).astype(jnp.bfloat16)
    w_flat = w_flat.reshape(k * k * Cin, Cout)

    conv_kernel = functools.partial(_conv_stats_kernel, k=k, h=H, w=W)
    y, stats = pl.pallas_call(
        conv_kernel,
        out_shape=(jax.ShapeDtypeStruct((N, Cout, rows), jnp.bfloat16),
                   jax.ShapeDtypeStruct((N // b1, 16, Cout), jnp.float32)),
        grid=(N // b1,),
        in_specs=[pl.BlockSpec((b1, H, W, Cin), lambda n: (n, 0, 0, 0)),
                  pl.BlockSpec((k * k * Cin, Cout), lambda n: (0, 0))],
        out_specs=(pl.BlockSpec((b1, Cout, rows), lambda n: (n, 0, 0)),
                   pl.BlockSpec((1, 16, Cout), lambda n: (n, 0, 0))),
        scratch_shapes=[pltpu.VMEM((Hp, Wp, Cin), jnp.bfloat16)],
        compiler_params=pltpu.CompilerParams(
            dimension_semantics=("parallel",),
            allow_input_fusion=[True, False],
            vmem_limit_bytes=_VMEM_LIMIT),
    )(xp, w_flat)

    bn_kernel = functools.partial(_bn_mish_kernel, count=float(N * rows),
                                  eps=eps)
    out_flat = pl.pallas_call(
        bn_kernel,
        out_shape=jax.ShapeDtypeStruct((N, Cout, rows), jnp.float32),
        grid=(N // b2,),
        in_specs=[pl.BlockSpec((b2, Cout, rows), lambda n: (n, 0, 0)),
                  pl.BlockSpec((N // b1, 16, Cout), lambda n: (0, 0, 0)),
                  pl.BlockSpec((1, Cout), lambda n: (0, 0)),
                  pl.BlockSpec((1, Cout), lambda n: (0, 0))],
        out_specs=pl.BlockSpec((b2, Cout, rows), lambda n: (n, 0, 0)),
        compiler_params=pltpu.CompilerParams(
            dimension_semantics=("parallel",),
            vmem_limit_bytes=_VMEM_LIMIT),
    )(y, stats, gamma.reshape(1, Cout).astype(jnp.float32),
      beta.reshape(1, Cout).astype(jnp.float32))

    return out_flat.reshape(N, Cout, H, W)


# speedup vs baseline: 1.2187x; 1.2187x over previous
import functools

import jax
import jax.numpy as jnp
from jax.experimental import pallas as pl
from jax.experimental.pallas import tpu as pltpu

_VMEM_LIMIT = 48 * 1024 * 1024
_B1 = 2
_B2 = 4


def _conv_stats_kernel(x_ref, w_ref, y_ref, stats_ref, slab_ref, *, k, h, w):
    rows = h * w
    ones8 = jnp.ones((8, rows), jnp.float32)
    s_tot = jnp.zeros((8, w_ref.shape[1]), jnp.float32)
    q_tot = jnp.zeros((8, w_ref.shape[1]), jnp.float32)
    slab_ref[...] = jnp.zeros_like(slab_ref)
    for b in range(x_ref.shape[0]):
        slab_ref[k // 2:k // 2 + h, k // 2:k // 2 + w, :] = x_ref[b]
        slab = slab_ref[...]
        cols = [slab[di:di + h, dj:dj + w, :].reshape(rows, -1)
                for di in range(k) for dj in range(k)]
        xcol = jnp.concatenate(cols, axis=1)
        acc = jnp.dot(xcol, w_ref[...],
                      preferred_element_type=jnp.float32)
        s_tot = s_tot + jnp.dot(ones8, acc,
                                preferred_element_type=jnp.float32)
        q_tot = q_tot + jnp.dot(ones8, acc * acc,
                                preferred_element_type=jnp.float32)
        y_ref[b] = acc.T.astype(jnp.bfloat16)
    stats_ref[0] = jnp.concatenate([s_tot, q_tot], axis=0)


def _bn_mish_kernel(y_ref, stats_ref, gamma_ref, beta_ref, o_ref,
                    *, count, eps):
    s = jnp.sum(stats_ref[:, 0, :], axis=0)
    q = jnp.sum(stats_ref[:, 8, :], axis=0)
    mean = s / count
    var = jnp.maximum(q / count - mean * mean, 0.0)
    inv_std = jax.lax.rsqrt(var + eps)
    g = gamma_ref[0]
    scale = (g * inv_std).reshape(1, -1, 1)
    shift = (beta_ref[0] - mean * g * inv_std).reshape(1, -1, 1)
    z = y_ref[...].astype(jnp.float32) * scale + shift
    t = jnp.exp(jnp.minimum(z, 20.0))
    u = t * (t + 2.0)
    mish = z * u * pl.reciprocal(u + 2.0, approx=True)
    o_ref[...] = jnp.where(z > 20.0, z, mish).astype(o_ref.dtype)


def kernel(x_nchw, conv_w, gamma, beta):
    eps = 1e-5
    N, Cin, H, W = x_nchw.shape
    Cout, _, k, _ = conv_w.shape
    p = k // 2
    rows = H * W
    Hp, Wp = H + 2 * p, W + 2 * p
    b1 = _B1 if N % _B1 == 0 else 1
    b2 = _B2 if N % _B2 == 0 else 1

    xp = jnp.transpose(x_nchw, (0, 2, 3, 1)).astype(jnp.bfloat16)

    w_flat = jnp.transpose(conv_w, (2, 3, 1, 0)).astype(jnp.bfloat16)
    w_flat = w_flat.reshape(k * k * Cin, Cout)

    conv_kernel = functools.partial(_conv_stats_kernel, k=k, h=H, w=W)
    y, stats = pl.pallas_call(
        conv_kernel,
        out_shape=(jax.ShapeDtypeStruct((N, Cout, rows), jnp.bfloat16),
                   jax.ShapeDtypeStruct((N // b1, 16, Cout), jnp.float32)),
        grid=(N // b1,),
        in_specs=[pl.BlockSpec((b1, H, W, Cin), lambda n: (n, 0, 0, 0)),
                  pl.BlockSpec((k * k * Cin, Cout), lambda n: (0, 0))],
        out_specs=(pl.BlockSpec((b1, Cout, rows), lambda n: (n, 0, 0)),
                   pl.BlockSpec((1, 16, Cout), lambda n: (n, 0, 0))),
        scratch_shapes=[pltpu.VMEM((Hp, Wp, Cin), jnp.bfloat16)],
        compiler_params=pltpu.CompilerParams(
            dimension_semantics=("parallel",),
            allow_input_fusion=[True, False],
            vmem_limit_bytes=_VMEM_LIMIT),
    )(xp, w_flat)

    bn_kernel = functools.partial(_bn_mish_kernel, count=float(N * rows),
                                  eps=eps)
    out_flat = pl.pallas_call(
        bn_kernel,
        out_shape=jax.ShapeDtypeStruct((N, Cout, rows), jnp.float32),
        grid=(N // b2,),
        in_specs=[pl.BlockSpec((b2, Cout, rows), lambda n: (n, 0, 0)),
                  pl.BlockSpec((N // b1, 16, Cout), lambda n: (0, 0, 0)),
                  pl.BlockSpec((1, Cout), lambda n: (0, 0)),
                  pl.BlockSpec((1, Cout), lambda n: (0, 0))],
        out_specs=pl.BlockSpec((b2, Cout, rows), lambda n: (n, 0, 0)),
        compiler_params=pltpu.CompilerParams(
            dimension_semantics=("parallel",),
            vmem_limit_bytes=_VMEM_LIMIT),
    )(y, stats, gamma.reshape(1, Cout).astype(jnp.float32),
      beta.reshape(1, Cout).astype(jnp.float32))

    return out_flat.reshape(N, Cout, H, W)
